# baseline (speedup 1.0000x reference)
import jax
import jax.numpy as jnp
from jax.experimental import pallas as pl
from jax.experimental.pallas import tpu as pltpu

N, C, T = 64, 256, 512
K = 8
ALPHA = 0.8
B = 8


def _dyn_graph_body(gamma_ref, x_ref, ap_ref, eye_ref, xo_ref, ao_ref):
    xv = x_ref[...]
    eye = eye_ref[...]
    gamma = gamma_ref[0]

    acc = jax.lax.dot_general(
        xv, xv, (((2,), (2,)), ((0,), (0,))),
        preferred_element_type=jnp.float32)

    mean = jnp.sum(xv, axis=2, keepdims=True) * (1.0 / T)
    d = jnp.sum(acc * eye, axis=2, keepdims=True)
    var = (d - (mean * mean) * T) * (1.0 / (T - 1))
    sinv = 1.0 / (jnp.sqrt(var) + 1e-06)

    a = sinv * (T ** -0.5)
    q = mean * sinv
    aT = jnp.swapaxes(a, 1, 2)
    qT = jnp.swapaxes(q, 1, 2)
    A = jnp.maximum(acc * (a * aT) - q * qT, 0.0)

    mx = jnp.maximum(A[..., : C // 2], A[..., C // 2:])
    mn = jnp.minimum(A[..., : C // 2], A[..., C // 2:])
    v = [jnp.max(mx, axis=2, keepdims=True)]
    for _ in range(K - 1):
        v.append(jnp.max(jnp.where(mx < v[-1], mx, -1.0),
                         axis=2, keepdims=True))
    w = [jnp.max(mn, axis=2, keepdims=True)]
    for _ in range(K // 2 - 1):
        w.append(jnp.max(jnp.where(mn < w[-1], mn, -1.0),
                         axis=2, keepdims=True))
    thr = jnp.maximum(v[7], jnp.minimum(v[6], w[0]))
    thr = jnp.maximum(thr, jnp.minimum(v[5], w[1]))
    thr = jnp.maximum(thr, jnp.minimum(v[4], w[2]))
    thr = jnp.maximum(thr, jnp.minimum(v[3], w[3]))

    mrow = jnp.where(A >= thr, A, 0.0)
    mcol = jnp.where(A >= jnp.swapaxes(thr, 1, 2), A, 0.0)
    S = 0.5 * (mrow + mcol) + eye

    deg = jnp.sum(S, axis=2, keepdims=True) + 1e-06
    rdeg = (1.0 - ALPHA) / deg
    A_out = ALPHA * ap_ref[...] + rdeg * S
    ao_ref[...] = A_out

    z = jax.lax.dot_general(
        gamma * A_out, xv, (((2,), (1,)), ((0,), (0,))),
        preferred_element_type=jnp.float32)
    xo_ref[...] = xv + z


def kernel(x, A_prev, gamma):
    gamma_arr = jnp.reshape(gamma.astype(jnp.float32), (1,))
    eye = jnp.eye(C, dtype=jnp.float32)[None]
    grid_spec = pltpu.PrefetchScalarGridSpec(
        num_scalar_prefetch=1,
        grid=(N // B,),
        in_specs=[
            pl.BlockSpec((B, C, T), lambda i, g: (i, 0, 0)),
            pl.BlockSpec((B, C, C), lambda i, g: (i, 0, 0)),
            pl.BlockSpec((1, C, C), lambda i, g: (0, 0, 0)),
        ],
        out_specs=[
            pl.BlockSpec((B, C, T), lambda i, g: (i, 0, 0)),
            pl.BlockSpec((B, C, C), lambda i, g: (i, 0, 0)),
        ],
    )
    x_out, A_out = pl.pallas_call(
        _dyn_graph_body,
        grid_spec=grid_spec,
        out_shape=[
            jax.ShapeDtypeStruct((N, C, T), jnp.float32),
            jax.ShapeDtypeStruct((N, C, C), jnp.float32),
        ],
        compiler_params=pltpu.CompilerParams(
            dimension_semantics=("parallel",),
        ),
    )(gamma_arr, x, A_prev, eye)
    return (x_out, A_out)


# speedup vs baseline: 1.2691x; 1.2691x over previous
import jax
import jax.numpy as jnp
from jax.experimental import pallas as pl
from jax.experimental.pallas import tpu as pltpu

N, C, T = 64, 256, 512
K = 8
ALPHA = 0.8
B = 8


def _dyn_graph_body(gamma_ref, x_ref, ap_ref, eye_ref, xo_ref, ao_ref):
    xv = x_ref[...]
    eye = eye_ref[...]
    gamma = gamma_ref[0]

    mean = jnp.sum(xv, axis=2, keepdims=True) * (1.0 / T)
    d = jnp.sum(xv * xv, axis=2, keepdims=True)
    var = (d - (mean * mean) * T) * (1.0 / (T - 1))
    sn = (1.0 / (jnp.sqrt(var) + 1e-06)) * (T ** -0.5)

    xn = (xv - mean) * sn
    A = jnp.maximum(
        jax.lax.dot_general(
            xn, xn, (((2,), (2,)), ((0,), (0,))),
            preferred_element_type=jnp.float32), 0.0)

    thr = jnp.max(A, axis=2, keepdims=True)
    for _ in range(K - 1):
        thr = jnp.max(jnp.where(A < thr, A, -1.0), axis=2, keepdims=True)

    mrow = jnp.where(A >= thr, A, 0.0)
    mcol = jnp.where(A >= jnp.swapaxes(thr, 1, 2), A, 0.0)
    S = 0.5 * (mrow + mcol) + eye

    deg = jnp.sum(S, axis=2, keepdims=True) + 1e-06
    rdeg = (1.0 - ALPHA) / deg
    A_out = ALPHA * ap_ref[...] + rdeg * S
    ao_ref[...] = A_out

    z = jax.lax.dot_general(
        gamma * A_out, xv, (((2,), (1,)), ((0,), (0,))),
        preferred_element_type=jnp.float32)
    xo_ref[...] = xv + z


def kernel(x, A_prev, gamma):
    gamma_arr = jnp.reshape(gamma.astype(jnp.float32), (1,))
    eye = jnp.eye(C, dtype=jnp.float32)[None]
    grid_spec = pltpu.PrefetchScalarGridSpec(
        num_scalar_prefetch=1,
        grid=(N // B,),
        in_specs=[
            pl.BlockSpec((B, C, T), lambda i, g: (i, 0, 0)),
            pl.BlockSpec((B, C, C), lambda i, g: (i, 0, 0)),
            pl.BlockSpec((1, C, C), lambda i, g: (0, 0, 0)),
        ],
        out_specs=[
            pl.BlockSpec((B, C, T), lambda i, g: (i, 0, 0)),
            pl.BlockSpec((B, C, C), lambda i, g: (i, 0, 0)),
        ],
    )
    x_out, A_out = pl.pallas_call(
        _dyn_graph_body,
        grid_spec=grid_spec,
        out_shape=[
            jax.ShapeDtypeStruct((N, C, T), jnp.float32),
            jax.ShapeDtypeStruct((N, C, C), jnp.float32),
        ],
        compiler_params=pltpu.CompilerParams(
            dimension_semantics=("parallel",),
        ),
    )(gamma_arr, x, A_prev, eye)
    return (x_out, A_out)


# speedup vs baseline: 1.2728x; 1.0029x over previous
import jax
import jax.numpy as jnp
from jax.experimental import pallas as pl
from jax.experimental.pallas import tpu as pltpu

N, C, T = 64, 256, 512
K = 8
ALPHA = 0.8
B = 8


def _dyn_graph_body(gamma_ref, x_ref, ap_ref, eye_ref, xo_ref, ao_ref):
    xv = x_ref[...]
    eye = eye_ref[...]
    gamma = gamma_ref[0]

    mean = jnp.sum(xv, axis=2, keepdims=True) * (1.0 / T)
    d = jnp.sum(xv * xv, axis=2, keepdims=True)
    var = (d - (mean * mean) * T) * (1.0 / (T - 1))
    sn = (1.0 / (jnp.sqrt(var) + 1e-06)) * (T ** -0.5)

    xn = (xv - mean) * sn
    A = jnp.maximum(
        jax.lax.dot_general(
            xn, xn, (((2,), (2,)), ((0,), (0,))),
            preferred_element_type=jnp.float32), 0.0)

    thr = jnp.max(A, axis=2, keepdims=True)
    for _ in range(K - 1):
        thr = jnp.max(jnp.where(A < thr, A, -1.0), axis=2, keepdims=True)

    mrow = jnp.where(A >= thr, A, 0.0)
    mcol = jnp.where(A >= jnp.swapaxes(thr, 1, 2), A, 0.0)
    S = 0.5 * (mrow + mcol) + eye

    deg = jnp.sum(S, axis=2, keepdims=True) + 1e-06
    rdeg = (1.0 - ALPHA) / deg
    A_out = ALPHA * ap_ref[...] + rdeg * S
    ao_ref[...] = A_out

    z = jax.lax.dot_general(
        gamma * A_out, xv, (((2,), (1,)), ((0,), (0,))),
        preferred_element_type=jnp.float32)
    xo_ref[...] = xv + z


def kernel(x, A_prev, gamma):
    gamma_arr = jnp.reshape(gamma.astype(jnp.float32), (1,))
    eye = jnp.eye(C, dtype=jnp.float32)[None]
    grid_spec = pltpu.PrefetchScalarGridSpec(
        num_scalar_prefetch=1,
        grid=(N // B,),
        in_specs=[
            pl.BlockSpec((B, C, T), lambda i, g: (i, 0, 0)),
            pl.BlockSpec((B, C, C), lambda i, g: (i, 0, 0)),
            pl.BlockSpec((1, C, C), lambda i, g: (0, 0, 0)),
        ],
        out_specs=[
            pl.BlockSpec((B, C, T), lambda i, g: (i, 0, 0)),
            pl.BlockSpec((B, C, C), lambda i, g: (i, 0, 0)),
        ],
    )
    x_out, A_out = pl.pallas_call(
        _dyn_graph_body,
        grid_spec=grid_spec,
        out_shape=[
            jax.ShapeDtypeStruct((N, C, T), jnp.float32),
            jax.ShapeDtypeStruct((N, C, C), jnp.float32),
        ],
        compiler_params=pltpu.CompilerParams(
            dimension_semantics=("arbitrary",),
        ),
    )(gamma_arr, x, A_prev, eye)
    return (x_out, A_out)
